# baseline (device time: 151949 ns/iter reference)
import jax
import jax.numpy as jnp
from jax import lax
from jax.experimental import pallas as pl
from jax.experimental.pallas import tpu as pltpu

N_DEV = 32
B, SQ, SKV, D = 2, 256, 256, 512
H_LOC, DH = 4, 64
HD_LOC = H_LOC * DH
ROWS = B * SQ
CHUNK = ROWS // N_DEV


def kernel(x, Wq, K_ext, V_ext, Wo):
    idx = lax.axis_index("i")
    wq_blk = lax.dynamic_slice(Wq, (0, idx * HD_LOC), (D, HD_LOC))
    wo_blk = lax.dynamic_slice(Wo, (idx * HD_LOC, 0), (HD_LOC, D))

    def body(x_ref, wq_ref, k_ref, v_ref, wo_ref, out_ref,
             acc_ref, rs_buf, rs_send, rs_recv, ag_send, ag_recv):
        my = lax.axis_index("i")
        right = jnp.remainder(my + 1, N_DEV)
        left = jnp.remainder(my + N_DEV - 1, N_DEV)

        qi = lax.broadcasted_iota(jnp.int32, (SQ, SKV), 0) // 64
        kj = lax.broadcasted_iota(jnp.int32, (SQ, SKV), 1) // 64
        mask = (qi == kj) | ((kj % 4) == (qi % 4))
        for b in range(B):
            q = jnp.dot(x_ref[b], wq_ref[:, :],
                        preferred_element_type=jnp.float32)
            part = jnp.zeros((SQ, D), jnp.float32)
            for h in range(H_LOC):
                qh = q[:, h * DH:(h + 1) * DH]
                kh = k_ref[b, :, h, :]
                vh = v_ref[b, :, h, :]
                s = jnp.dot(qh, kh.T,
                            preferred_element_type=jnp.float32) * 0.125
                s = jnp.where(mask, s, -1e9)
                w = jnp.exp(s - jnp.max(s, axis=1, keepdims=True))
                w = w / jnp.sum(w, axis=1, keepdims=True)
                ctx = jnp.dot(w, vh, preferred_element_type=jnp.float32)
                part = part + jnp.dot(ctx, wo_ref[h * DH:(h + 1) * DH, :],
                                      preferred_element_type=jnp.float32)
            acc_ref[pl.ds(b * SQ, SQ), :] = part

        bar = pltpu.get_barrier_semaphore()
        for nbr in (left, right):
            pl.semaphore_signal(bar, inc=1, device_id=(nbr,),
                                device_id_type=pl.DeviceIdType.MESH)
        pl.semaphore_wait(bar, 2)

        for s in range(N_DEV - 1):
            c_send = jnp.remainder(my + N_DEV - s, N_DEV)
            c_recv = jnp.remainder(my + N_DEV - 1 - s, N_DEV)
            rdma = pltpu.make_async_remote_copy(
                src_ref=acc_ref.at[pl.ds(c_send * CHUNK, CHUNK), :],
                dst_ref=rs_buf.at[s],
                send_sem=rs_send.at[s],
                recv_sem=rs_recv.at[s],
                device_id=(right,),
                device_id_type=pl.DeviceIdType.MESH,
            )
            rdma.start()
            rdma.wait()
            acc_ref[pl.ds(c_recv * CHUNK, CHUNK), :] = (
                acc_ref[pl.ds(c_recv * CHUNK, CHUNK), :] + rs_buf[s]
            )

        for s in range(N_DEV - 1):
            c_send = jnp.remainder(my + N_DEV + 1 - s, N_DEV)
            rdma = pltpu.make_async_remote_copy(
                src_ref=acc_ref.at[pl.ds(c_send * CHUNK, CHUNK), :],
                dst_ref=acc_ref.at[pl.ds(c_send * CHUNK, CHUNK), :],
                send_sem=ag_send.at[s],
                recv_sem=ag_recv.at[s],
                device_id=(right,),
                device_id_type=pl.DeviceIdType.MESH,
            )
            rdma.start()
            rdma.wait()

        for b in range(B):
            out_ref[b] = acc_ref[pl.ds(b * SQ, SQ), :]

    return pl.pallas_call(
        body,
        out_shape=jax.ShapeDtypeStruct((B, SQ, D), jnp.float32),
        in_specs=[pl.BlockSpec(memory_space=pltpu.VMEM)] * 5,
        out_specs=pl.BlockSpec(memory_space=pltpu.VMEM),
        scratch_shapes=[
            pltpu.VMEM((ROWS, D), jnp.float32),
            pltpu.VMEM((N_DEV - 1, CHUNK, D), jnp.float32),
            pltpu.SemaphoreType.DMA((N_DEV - 1,)),
            pltpu.SemaphoreType.DMA((N_DEV - 1,)),
            pltpu.SemaphoreType.DMA((N_DEV - 1,)),
            pltpu.SemaphoreType.DMA((N_DEV - 1,)),
        ],
        compiler_params=pltpu.CompilerParams(collective_id=0),
    )(x, wq_blk, K_ext, V_ext, wo_blk)


# device time: 46627 ns/iter; 3.2588x vs baseline; 3.2588x over previous
import jax
import jax.numpy as jnp
from jax import lax
from jax.experimental import pallas as pl
from jax.experimental.pallas import tpu as pltpu

N_DEV = 32
B, SQ, SKV, D = 2, 256, 256, 512
H_LOC, DH = 4, 64
HD_LOC = H_LOC * DH
ROWS = B * SQ
CHUNK = ROWS // N_DEV


def kernel(x, Wq, K_ext, V_ext, Wo):
    idx = lax.axis_index("i")
    wq_blk = lax.dynamic_slice(Wq, (0, idx * HD_LOC), (D, HD_LOC))
    wo_blk = lax.dynamic_slice(Wo, (idx * HD_LOC, 0), (HD_LOC, D))

    def body(x_ref, wq_ref, k_ref, v_ref, wo_ref, out_ref,
             acc_ref, rs_buf, rs_send, rs_recv, ag_send, ag_recv):
        my = lax.axis_index("i")

        qi = lax.broadcasted_iota(jnp.int32, (SQ, SKV), 0) // 64
        kj = lax.broadcasted_iota(jnp.int32, (SQ, SKV), 1) // 64
        mask = (qi == kj) | ((kj % 4) == (qi % 4))
        for b in range(B):
            q = jnp.dot(x_ref[b], wq_ref[:, :],
                        preferred_element_type=jnp.float32)
            part = jnp.zeros((SQ, D), jnp.float32)
            for h in range(H_LOC):
                qh = q[:, h * DH:(h + 1) * DH]
                kh = k_ref[b, :, h, :]
                vh = v_ref[b, :, h, :]
                s = jnp.dot(qh, kh.T,
                            preferred_element_type=jnp.float32) * 0.125
                s = jnp.where(mask, s, -1e9)
                w = jnp.exp(s - jnp.max(s, axis=1, keepdims=True))
                w = w / jnp.sum(w, axis=1, keepdims=True)
                ctx = jnp.dot(w, vh, preferred_element_type=jnp.float32)
                part = part + jnp.dot(ctx, wo_ref[h * DH:(h + 1) * DH, :],
                                      preferred_element_type=jnp.float32)
            acc_ref[pl.ds(b * SQ, SQ), :] = part

        bar = pltpu.get_barrier_semaphore()
        for d in range(1, N_DEV):
            pl.semaphore_signal(
                bar, inc=1,
                device_id=(jnp.remainder(my + d, N_DEV),),
                device_id_type=pl.DeviceIdType.MESH,
            )
        pl.semaphore_wait(bar, N_DEV - 1)

        rs_sends = []
        for d in range(1, N_DEV):
            tgt = jnp.remainder(my + d, N_DEV)
            rdma = pltpu.make_async_remote_copy(
                src_ref=acc_ref.at[pl.ds(tgt * CHUNK, CHUNK), :],
                dst_ref=rs_buf.at[N_DEV - d],
                send_sem=rs_send.at[d],
                recv_sem=rs_recv.at[N_DEV - d],
                device_id=(tgt,),
                device_id_type=pl.DeviceIdType.MESH,
            )
            rdma.start()
            rs_sends.append(rdma)

        red = acc_ref[pl.ds(my * CHUNK, CHUNK), :]
        for e in range(1, N_DEV):
            recv = pltpu.make_async_remote_copy(
                src_ref=acc_ref.at[pl.ds(my * CHUNK, CHUNK), :],
                dst_ref=rs_buf.at[e],
                send_sem=rs_send.at[e],
                recv_sem=rs_recv.at[e],
                device_id=(my,),
                device_id_type=pl.DeviceIdType.MESH,
            )
            recv.wait_recv()
            red = red + rs_buf[e]
        acc_ref[pl.ds(my * CHUNK, CHUNK), :] = red

        ag_sends = []
        for d in range(1, N_DEV):
            tgt = jnp.remainder(my + d, N_DEV)
            rdma = pltpu.make_async_remote_copy(
                src_ref=acc_ref.at[pl.ds(my * CHUNK, CHUNK), :],
                dst_ref=acc_ref.at[pl.ds(my * CHUNK, CHUNK), :],
                send_sem=ag_send.at[d],
                recv_sem=ag_recv.at[N_DEV - d],
                device_id=(tgt,),
                device_id_type=pl.DeviceIdType.MESH,
            )
            rdma.start()
            ag_sends.append(rdma)

        for e in range(1, N_DEV):
            src_dev = jnp.remainder(my + e, N_DEV)
            recv = pltpu.make_async_remote_copy(
                src_ref=acc_ref.at[pl.ds(my * CHUNK, CHUNK), :],
                dst_ref=acc_ref.at[pl.ds(src_dev * CHUNK, CHUNK), :],
                send_sem=ag_send.at[e],
                recv_sem=ag_recv.at[e],
                device_id=(my,),
                device_id_type=pl.DeviceIdType.MESH,
            )
            recv.wait_recv()

        for b in range(B):
            out_ref[b] = acc_ref[pl.ds(b * SQ, SQ), :]

        for rdma in rs_sends:
            rdma.wait_send()
        for rdma in ag_sends:
            rdma.wait_send()

    return pl.pallas_call(
        body,
        out_shape=jax.ShapeDtypeStruct((B, SQ, D), jnp.float32),
        in_specs=[pl.BlockSpec(memory_space=pltpu.VMEM)] * 5,
        out_specs=pl.BlockSpec(memory_space=pltpu.VMEM),
        scratch_shapes=[
            pltpu.VMEM((ROWS, D), jnp.float32),
            pltpu.VMEM((N_DEV, CHUNK, D), jnp.float32),
            pltpu.SemaphoreType.DMA((N_DEV,)),
            pltpu.SemaphoreType.DMA((N_DEV,)),
            pltpu.SemaphoreType.DMA((N_DEV,)),
            pltpu.SemaphoreType.DMA((N_DEV,)),
        ],
        compiler_params=pltpu.CompilerParams(collective_id=0),
    )(x, wq_blk, K_ext, V_ext, wo_blk)


# device time: 26274 ns/iter; 5.7832x vs baseline; 1.7746x over previous
import jax
import jax.numpy as jnp
from jax import lax
from jax.experimental import pallas as pl
from jax.experimental.pallas import tpu as pltpu

N_DEV = 32
B, SQ, SKV, D = 2, 256, 256, 512
H_LOC, DH = 4, 64
HD_LOC = H_LOC * DH
ROWS = B * SQ
CHUNK = ROWS // N_DEV
CPB = SQ // CHUNK
import os
PHASES = int(os.environ.get("KPHASES", "3"))
COMM_ABLATION = PHASES < 2


def kernel(x, Wq, K_ext, V_ext, Wo):
    idx = lax.axis_index("i")
    wq_blk = lax.dynamic_slice(Wq, (0, idx * HD_LOC), (D, HD_LOC))
    wo_blk = lax.dynamic_slice(Wo, (idx * HD_LOC, 0), (HD_LOC, D))

    def body(x_ref, wq_ref, k_ref, v_ref, wo_ref, out_ref,
             acc_ref, send_bf, rs_buf, ag_buf,
             rs_send, rs_recv, ag_send, ag_recv):
        my = lax.axis_index("i")

        if not COMM_ABLATION:
            bar = pltpu.get_barrier_semaphore()
            for d in range(1, N_DEV):
                pl.semaphore_signal(
                    bar, inc=1,
                    device_id=(jnp.remainder(my + d, N_DEV),),
                    device_id_type=pl.DeviceIdType.MESH,
                )
            pl.semaphore_wait(bar, N_DEV - 1)

        def make_rs(d):
            tgt = jnp.remainder(my + d, N_DEV)
            return pltpu.make_async_remote_copy(
                src_ref=send_bf.at[pl.ds(tgt * CHUNK, CHUNK), :],
                dst_ref=rs_buf.at[N_DEV - d],
                send_sem=rs_send.at[d],
                recv_sem=rs_recv.at[N_DEV - d],
                device_id=(tgt,),
                device_id_type=pl.DeviceIdType.MESH,
            ), tgt

        bf = jnp.bfloat16
        qi = lax.broadcasted_iota(jnp.int32, (SQ, SKV), 0) // 64
        kj = lax.broadcasted_iota(jnp.int32, (SQ, SKV), 1) // 64
        mask = (qi == kj) | ((kj % 4) == (qi % 4))
        wq_b = wq_ref[:, :].astype(bf)
        wo_b = wo_ref[:, :].astype(bf)
        for b in range(B):
            q = jnp.dot(x_ref[b].astype(bf), wq_b,
                        preferred_element_type=jnp.float32)
            part = jnp.zeros((SQ, D), jnp.float32)
            for h in range(H_LOC):
                qh = q[:, h * DH:(h + 1) * DH].astype(bf)
                kh = k_ref[b, :, h, :].astype(bf)
                vh = v_ref[b, :, h, :].astype(bf)
                s = jnp.dot(qh, kh.T,
                            preferred_element_type=jnp.float32) * 0.125
                s = jnp.where(mask, s, -1e9)
                w = jnp.exp(s - jnp.max(s, axis=1, keepdims=True))
                w = w / jnp.sum(w, axis=1, keepdims=True)
                ctx = jnp.dot(w.astype(bf), vh,
                              preferred_element_type=jnp.float32)
                part = part + jnp.dot(ctx.astype(bf),
                                      wo_b[h * DH:(h + 1) * DH, :],
                                      preferred_element_type=jnp.float32)
            acc_ref[pl.ds(b * SQ, SQ), :] = part
            send_bf[pl.ds(b * SQ, SQ), :] = part.astype(bf)

            if not COMM_ABLATION:
                for d in range(1, N_DEV):
                    rdma, tgt = make_rs(d)
                    in_batch = (tgt // CPB) == b

                    @pl.when(in_batch)
                    def _(rdma=rdma):
                        rdma.start()

        if COMM_ABLATION:
            for b in range(B):
                out_ref[b] = acc_ref[pl.ds(b * SQ, SQ), :]
            return

        for e in range(1, N_DEV):
            recv = pltpu.make_async_remote_copy(
                src_ref=send_bf.at[pl.ds(my * CHUNK, CHUNK), :],
                dst_ref=rs_buf.at[e],
                send_sem=rs_send.at[e],
                recv_sem=rs_recv.at[e],
                device_id=(my,),
                device_id_type=pl.DeviceIdType.MESH,
            )
            recv.wait_recv()
        red = acc_ref[pl.ds(my * CHUNK, CHUNK), :] + jnp.sum(
            rs_buf[1:, :, :].astype(jnp.float32), axis=0
        )
        my_b = my // CPB
        my_row = jnp.remainder(my, CPB) * CHUNK
        out_ref[my_b, pl.ds(my_row, CHUNK), :] = red
        send_bf[pl.ds(my * CHUNK, CHUNK), :] = red.astype(bf)

        if PHASES < 3:
            for d in range(1, N_DEV):
                rdma, _ = make_rs(d)
                rdma.wait_send()
            return
        ag_sends = []
        for d in range(1, N_DEV):
            tgt = jnp.remainder(my + d, N_DEV)
            rdma = pltpu.make_async_remote_copy(
                src_ref=send_bf.at[pl.ds(my * CHUNK, CHUNK), :],
                dst_ref=ag_buf.at[N_DEV - d],
                send_sem=ag_send.at[d],
                recv_sem=ag_recv.at[N_DEV - d],
                device_id=(tgt,),
                device_id_type=pl.DeviceIdType.MESH,
            )
            rdma.start()
            ag_sends.append(rdma)

        for e in range(1, N_DEV):
            recv = pltpu.make_async_remote_copy(
                src_ref=send_bf.at[pl.ds(my * CHUNK, CHUNK), :],
                dst_ref=ag_buf.at[e],
                send_sem=ag_send.at[e],
                recv_sem=ag_recv.at[e],
                device_id=(my,),
                device_id_type=pl.DeviceIdType.MESH,
            )
            recv.wait_recv()
            s_dev = jnp.remainder(my + e, N_DEV)
            s_b = s_dev // CPB
            s_row = jnp.remainder(s_dev, CPB) * CHUNK
            out_ref[s_b, pl.ds(s_row, CHUNK), :] = (
                ag_buf[e].astype(jnp.float32)
            )

        for d in range(1, N_DEV):
            rdma, _ = make_rs(d)
            rdma.wait_send()
        for rdma in ag_sends:
            rdma.wait_send()

    return pl.pallas_call(
        body,
        out_shape=jax.ShapeDtypeStruct((B, SQ, D), jnp.float32),
        in_specs=[pl.BlockSpec(memory_space=pltpu.VMEM)] * 5,
        out_specs=pl.BlockSpec(memory_space=pltpu.VMEM),
        scratch_shapes=[
            pltpu.VMEM((ROWS, D), jnp.float32),
            pltpu.VMEM((ROWS, D), jnp.bfloat16),
            pltpu.VMEM((N_DEV, CHUNK, D), jnp.bfloat16),
            pltpu.VMEM((N_DEV, CHUNK, D), jnp.bfloat16),
            pltpu.SemaphoreType.DMA((N_DEV,)),
            pltpu.SemaphoreType.DMA((N_DEV,)),
            pltpu.SemaphoreType.DMA((N_DEV,)),
            pltpu.SemaphoreType.DMA((N_DEV,)),
        ],
        compiler_params=(
            pltpu.CompilerParams()
            if COMM_ABLATION
            else pltpu.CompilerParams(collective_id=0)
        ),
    )(x, wq_blk, K_ext, V_ext, wo_blk)
